# baseline (speedup 1.0000x reference)
import functools

import jax
import jax.numpy as jnp
from jax import lax
from jax.experimental import pallas as pl
from jax.experimental.pallas import tpu as pltpu
from jax.experimental.pallas import tpu_sc as plsc

B, S, D = 1, 2048, 768
E, TOPK = 8, 2
I = 341
RT = 128
NT = 39
P = NT * RT
NC, NS = 2, 16
NW = NC * NS
TW = S // NW
CH = D // 16


def _router_body(x_ref, gw_ref, sw_ref, sd_ref, tri_ref,
                 shared_ref, wbc_ref, pos8_ref, texp_ref):
    xt = x_ref[...]
    logits = jnp.dot(xt, gw_ref[...], preferred_element_type=jnp.float32)
    lane = jax.lax.broadcasted_iota(jnp.int32, (S, E), 1)
    m = jnp.max(logits, axis=1, keepdims=True)
    ex = jnp.exp(logits - m)
    probs = ex / jnp.sum(ex, axis=1, keepdims=True)
    p1 = jnp.max(probs, axis=1, keepdims=True)
    sel1 = jnp.min(jnp.where(probs == p1, lane, E), axis=1, keepdims=True)
    probs2 = jnp.where(lane == sel1, -1.0, probs)
    p2 = jnp.max(probs2, axis=1, keepdims=True)
    sel2 = jnp.min(jnp.where(probs2 == p2, lane, E), axis=1, keepdims=True)
    wsum = p1 + p2 + 1e-8
    w1 = p1 / wsum
    w2 = p2 / wsum
    lane32 = jax.lax.broadcasted_iota(jnp.int32, (S, 32), 1)
    wbc_ref[...] = jnp.where(lane32 < 16, w1, w2)

    mask0 = (lane == sel1)
    mask1 = (lane == sel2)
    m01 = jnp.concatenate(
        [mask0.astype(jnp.bfloat16), mask1.astype(jnp.bfloat16)], axis=1)
    ranks = jnp.dot(tri_ref[...], m01,
                    preferred_element_type=jnp.float32)
    rank0 = ranks[:, :E]
    rank1 = ranks[:, E:]
    m0f = mask0.astype(jnp.float32)
    m1f = mask1.astype(jnp.float32)
    cnt0 = jnp.sum(m0f, axis=0, keepdims=True)
    cnt = cnt0 + jnp.sum(m1f, axis=0, keepdims=True)
    padded = jnp.ceil(cnt / RT) * RT
    er = jax.lax.broadcasted_iota(jnp.int32, (E, E), 0)
    ec = jax.lax.broadcasted_iota(jnp.int32, (E, E), 1)
    su = (er < ec).astype(jnp.float32)
    off = jnp.dot(padded, su, preferred_element_type=jnp.float32)
    slot0 = jnp.sum(m0f * (off + rank0), axis=1, keepdims=True)
    slot1 = jnp.sum(m1f * (off + rank1 + cnt0), axis=1, keepdims=True)
    pos8_ref[...] = jnp.where(lane == 0, slot0,
                              jnp.where(lane == 1, slot1, 0.0)).astype(
                                  jnp.int32)
    ti = jax.lax.broadcasted_iota(jnp.int32, (8, 128), 1).astype(jnp.float32)
    texp = jnp.zeros((8, 128), jnp.int32)
    for e in range(1, E):
        texp = texp + (ti * RT >= off[0, e]).astype(jnp.int32)
    texp_ref[...] = texp

    xb = xt.astype(jnp.bfloat16)
    gu = jnp.dot(xb, sw_ref[...], preferred_element_type=jnp.float32)
    g = gu[:, :I]
    u = gu[:, I:]
    h = (g * jax.nn.sigmoid(g)) * u
    shared_ref[...] = jnp.dot(h.astype(jnp.bfloat16), sd_ref[...],
                              preferred_element_type=jnp.float32)


@jax.jit
def _stage_a(x, gw, sw, sd, tri):
    return pl.pallas_call(
        _router_body,
        out_shape=(
            jax.ShapeDtypeStruct((S, D), jnp.float32),
            jax.ShapeDtypeStruct((S, 32), jnp.float32),
            jax.ShapeDtypeStruct((S, E), jnp.int32),
            jax.ShapeDtypeStruct((8, 128), jnp.int32),
        ),
    )(x, gw, sw, sd, tri)


def _dispatch_body(x_hbm, pos0_hbm, pos1_hbm, xs_hbm, xrow, idx, sem):
    wid = lax.axis_index("s") * NC + lax.axis_index("c")
    base = wid * TW
    pltpu.sync_copy(x_hbm.at[pl.ds(base, TW)], xrow)
    pltpu.sync_copy(pos0_hbm.at[pl.ds(base, TW)], idx)
    pltpu.async_copy(xrow, xs_hbm.at[idx], sem).wait()
    pltpu.sync_copy(pos1_hbm.at[pl.ds(base, TW)], idx)
    pltpu.async_copy(xrow, xs_hbm.at[idx], sem).wait()


def _ffn_body(texp_ref, xs_ref, ew_ref, ed_ref, ys_ref):
    del texp_ref
    xb = xs_ref[...].astype(jnp.bfloat16)
    gu = jnp.dot(xb, ew_ref[0], preferred_element_type=jnp.float32)
    g = gu[:, :I]
    u = gu[:, I:]
    h = (g * jax.nn.sigmoid(g)) * u
    ys_ref[...] = jnp.dot(h.astype(jnp.bfloat16), ed_ref[0],
                          preferred_element_type=jnp.float32)


@jax.jit
def _stage_c(texp, xs, ew, ed):
    grid_spec = pltpu.PrefetchScalarGridSpec(
        num_scalar_prefetch=1,
        grid=(NT,),
        in_specs=[
            pl.BlockSpec((RT, D), lambda i, t: (i, 0)),
            pl.BlockSpec((1, D, 2 * I), lambda i, t: (t[i], 0, 0)),
            pl.BlockSpec((1, I, D), lambda i, t: (t[i], 0, 0)),
        ],
        out_specs=pl.BlockSpec((RT, D), lambda i, t: (i, 0)),
    )
    return pl.pallas_call(
        _ffn_body,
        grid_spec=grid_spec,
        out_shape=jax.ShapeDtypeStruct((P, D), jnp.float32),
    )(texp, xs, ew, ed)


def _combine_body(shared_hbm, ys_hbm, pos0_hbm, pos1_hbm, wb0_hbm, wb1_hbm,
                  out_hbm, acc, ybuf, idx, wbv, sem):
    wid = lax.axis_index("s") * NC + lax.axis_index("c")
    base = wid * TW
    pltpu.sync_copy(shared_hbm.at[pl.ds(base, TW)], acc)
    for pos_hbm, wb_hbm in ((pos0_hbm, wb0_hbm), (pos1_hbm, wb1_hbm)):
        pltpu.sync_copy(pos_hbm.at[pl.ds(base, TW)], idx)
        pltpu.sync_copy(wb_hbm.at[pl.ds(base, TW)], wbv)
        pltpu.async_copy(ys_hbm.at[idx], ybuf, sem).wait()

        def body(t, carry):
            w = wbv[t]
            for c in range(CH):
                sl = pl.ds(c * 16, 16)
                acc[t, sl] = acc[t, sl] + w * ybuf[t, sl]
            return carry

        lax.fori_loop(0, TW, body, 0)
    pltpu.sync_copy(acc, out_hbm.at[pl.ds(base, TW)])


def kernel(hidden_states, shared_gate_up_w, shared_down_w, expert_gate_up,
           expert_down, gate_weight):
    b, s, d = hidden_states.shape
    x = hidden_states.reshape(s, d)
    gw = gate_weight.T
    sw = shared_gate_up_w.T.astype(jnp.bfloat16)
    sd = shared_down_w.T.astype(jnp.bfloat16)
    ew = expert_gate_up.astype(jnp.bfloat16)
    ed = expert_down.astype(jnp.bfloat16)
    tri = jnp.tril(jnp.ones((s, s), jnp.bfloat16), -1)

    mesh = plsc.VectorSubcoreMesh(core_axis_name="c", subcore_axis_name="s")
    dispatch = pl.kernel(
        _dispatch_body,
        out_type=jax.ShapeDtypeStruct((P, D), jnp.float32),
        mesh=mesh,
        scratch_types=[
            pltpu.VMEM((TW, D), jnp.float32),
            pltpu.VMEM((TW,), jnp.int32),
            pltpu.SemaphoreType.DMA,
        ],
    )
    combine = pl.kernel(
        _combine_body,
        out_type=jax.ShapeDtypeStruct((S, D), jnp.float32),
        mesh=mesh,
        scratch_types=[
            pltpu.VMEM((TW, D), jnp.float32),
            pltpu.VMEM((TW, D), jnp.float32),
            pltpu.VMEM((TW,), jnp.int32),
            pltpu.VMEM((TW, 16), jnp.float32),
            pltpu.SemaphoreType.DMA,
        ],
    )

    shared_out, wbc, pos8, texp8 = _stage_a(x, gw, sw, sd, tri)
    posT = jnp.transpose(pos8[:, :TOPK])
    wb0 = wbc[:, :16]
    wb1 = wbc[:, 16:]
    texp = texp8[0, :NT]

    xs = dispatch(x, posT[0], posT[1])
    ys = _stage_c(texp, xs, ew, ed)
    out = combine(shared_out, ys, posT[0], posT[1], wb0, wb1)
    return out.reshape(b, s, d)


# speedup vs baseline: 1.0071x; 1.0071x over previous
import functools

import jax
import jax.numpy as jnp
from jax import lax
from jax.experimental import pallas as pl
from jax.experimental.pallas import tpu as pltpu
from jax.experimental.pallas import tpu_sc as plsc

B, S, D = 1, 2048, 768
E, TOPK = 8, 2
I = 341
RT = 256
NT = 23
P = NT * RT
NC, NS = 2, 16
NW = NC * NS
TW = S // NW
CH = D // 16


def _router_body(x_ref, gw_ref, tri_ref,
                 wbc_ref, pos8_ref, texp_ref):
    xt = x_ref[...]
    logits = jnp.dot(xt, gw_ref[...], preferred_element_type=jnp.float32)
    lane = jax.lax.broadcasted_iota(jnp.int32, (S, E), 1)
    m = jnp.max(logits, axis=1, keepdims=True)
    ex = jnp.exp(logits - m)
    probs = ex / jnp.sum(ex, axis=1, keepdims=True)
    p1 = jnp.max(probs, axis=1, keepdims=True)
    sel1 = jnp.min(jnp.where(probs == p1, lane, E), axis=1, keepdims=True)
    probs2 = jnp.where(lane == sel1, -1.0, probs)
    p2 = jnp.max(probs2, axis=1, keepdims=True)
    sel2 = jnp.min(jnp.where(probs2 == p2, lane, E), axis=1, keepdims=True)
    wsum = p1 + p2 + 1e-8
    w1 = p1 / wsum
    w2 = p2 / wsum
    lane32 = jax.lax.broadcasted_iota(jnp.int32, (S, 32), 1)
    wbc_ref[...] = jnp.where(lane32 < 16, w1, w2)

    mask0 = (lane == sel1)
    mask1 = (lane == sel2)
    m01 = jnp.concatenate(
        [mask0.astype(jnp.bfloat16), mask1.astype(jnp.bfloat16)], axis=1)
    ranks = jnp.dot(tri_ref[...], m01,
                    preferred_element_type=jnp.float32)
    rank0 = ranks[:, :E]
    rank1 = ranks[:, E:]
    m0f = mask0.astype(jnp.float32)
    m1f = mask1.astype(jnp.float32)
    cnt0 = jnp.sum(m0f, axis=0, keepdims=True)
    cnt = cnt0 + jnp.sum(m1f, axis=0, keepdims=True)
    padded = jnp.ceil(cnt / RT) * RT
    er = jax.lax.broadcasted_iota(jnp.int32, (E, E), 0)
    ec = jax.lax.broadcasted_iota(jnp.int32, (E, E), 1)
    su = (er < ec).astype(jnp.float32)
    off = jnp.dot(padded, su, preferred_element_type=jnp.float32)
    slot0 = jnp.sum(m0f * (off + rank0), axis=1, keepdims=True)
    slot1 = jnp.sum(m1f * (off + rank1 + cnt0), axis=1, keepdims=True)
    pos8_ref[...] = jnp.where(lane == 0, slot0,
                              jnp.where(lane == 1, slot1, 0.0)).astype(
                                  jnp.int32)
    ti = jax.lax.broadcasted_iota(jnp.int32, (8, 128), 1).astype(jnp.float32)
    texp = jnp.zeros((8, 128), jnp.int32)
    for e in range(1, E):
        texp = texp + (ti * RT >= off[0, e]).astype(jnp.int32)
    texp_ref[...] = texp


@jax.jit
def _stage_a(x, gw, tri):
    return pl.pallas_call(
        _router_body,
        out_shape=(
            jax.ShapeDtypeStruct((S, 32), jnp.float32),
            jax.ShapeDtypeStruct((S, E), jnp.int32),
            jax.ShapeDtypeStruct((8, 128), jnp.int32),
        ),
    )(x, gw, tri)


def _shared_body(x_ref, sw_ref, sd_ref, shared_ref):
    xb = x_ref[...].astype(jnp.bfloat16)
    gu = jnp.dot(xb, sw_ref[...], preferred_element_type=jnp.float32)
    g = gu[:, :I]
    u = gu[:, I:]
    h = (g * jax.nn.sigmoid(g)) * u
    shared_ref[...] = jnp.dot(h.astype(jnp.bfloat16), sd_ref[...],
                              preferred_element_type=jnp.float32)


@jax.jit
def _stage_shared(x, sw, sd):
    return pl.pallas_call(
        _shared_body,
        out_shape=jax.ShapeDtypeStruct((S, D), jnp.float32),
    )(x, sw, sd)


def _dispatch_body(x_hbm, pos0_hbm, pos1_hbm, xs_hbm, xrow, idx, sem):
    wid = lax.axis_index("s") * NC + lax.axis_index("c")
    base = wid * TW
    pltpu.sync_copy(x_hbm.at[pl.ds(base, TW)], xrow)
    pltpu.sync_copy(pos0_hbm.at[pl.ds(base, TW)], idx)
    pltpu.async_copy(xrow, xs_hbm.at[idx], sem).wait()
    pltpu.sync_copy(pos1_hbm.at[pl.ds(base, TW)], idx)
    pltpu.async_copy(xrow, xs_hbm.at[idx], sem).wait()


def _ffn_body(texp_ref, xs_ref, ew_ref, ed_ref, ys_ref):
    del texp_ref
    xb = xs_ref[...].astype(jnp.bfloat16)
    gu = jnp.dot(xb, ew_ref[0], preferred_element_type=jnp.float32)
    g = gu[:, :I]
    u = gu[:, I:]
    h = (g * jax.nn.sigmoid(g)) * u
    ys_ref[...] = jnp.dot(h.astype(jnp.bfloat16), ed_ref[0],
                          preferred_element_type=jnp.float32)


@jax.jit
def _stage_c(texp, xs, ew, ed):
    grid_spec = pltpu.PrefetchScalarGridSpec(
        num_scalar_prefetch=1,
        grid=(NT,),
        in_specs=[
            pl.BlockSpec((RT, D), lambda i, t: (i, 0)),
            pl.BlockSpec((1, D, 2 * I), lambda i, t: (t[i], 0, 0)),
            pl.BlockSpec((1, I, D), lambda i, t: (t[i], 0, 0)),
        ],
        out_specs=pl.BlockSpec((RT, D), lambda i, t: (i, 0)),
    )
    return pl.pallas_call(
        _ffn_body,
        grid_spec=grid_spec,
        out_shape=jax.ShapeDtypeStruct((P, D), jnp.float32),
    )(texp, xs, ew, ed)


def _combine_body(shared_hbm, ys_hbm, pos0_hbm, pos1_hbm, wb0_hbm, wb1_hbm,
                  out_hbm, acc, ybuf, idx, wbv, sem):
    wid = lax.axis_index("s") * NC + lax.axis_index("c")
    base = wid * TW
    pltpu.sync_copy(shared_hbm.at[pl.ds(base, TW)], acc)
    for pos_hbm, wb_hbm in ((pos0_hbm, wb0_hbm), (pos1_hbm, wb1_hbm)):
        pltpu.sync_copy(pos_hbm.at[pl.ds(base, TW)], idx)
        pltpu.sync_copy(wb_hbm.at[pl.ds(base, TW)], wbv)
        pltpu.async_copy(ys_hbm.at[idx], ybuf, sem).wait()

        def body(t, carry):
            w = wbv[t]
            for c in range(CH):
                sl = pl.ds(c * 16, 16)
                acc[t, sl] = acc[t, sl] + w * ybuf[t, sl]
            return carry

        lax.fori_loop(0, TW, body, 0)
    pltpu.sync_copy(acc, out_hbm.at[pl.ds(base, TW)])


def kernel(hidden_states, shared_gate_up_w, shared_down_w, expert_gate_up,
           expert_down, gate_weight):
    b, s, d = hidden_states.shape
    x = hidden_states.reshape(s, d)
    gw = gate_weight.T
    sw = shared_gate_up_w.T.astype(jnp.bfloat16)
    sd = shared_down_w.T.astype(jnp.bfloat16)
    ew = expert_gate_up.astype(jnp.bfloat16)
    ed = expert_down.astype(jnp.bfloat16)
    tri = jnp.tril(jnp.ones((s, s), jnp.bfloat16), -1)

    mesh = plsc.VectorSubcoreMesh(core_axis_name="c", subcore_axis_name="s")
    dispatch = pl.kernel(
        _dispatch_body,
        out_type=jax.ShapeDtypeStruct((P, D), jnp.float32),
        mesh=mesh,
        scratch_types=[
            pltpu.VMEM((TW, D), jnp.float32),
            pltpu.VMEM((TW,), jnp.int32),
            pltpu.SemaphoreType.DMA,
        ],
    )
    combine = pl.kernel(
        _combine_body,
        out_type=jax.ShapeDtypeStruct((S, D), jnp.float32),
        mesh=mesh,
        scratch_types=[
            pltpu.VMEM((TW, D), jnp.float32),
            pltpu.VMEM((TW, D), jnp.float32),
            pltpu.VMEM((TW,), jnp.int32),
            pltpu.VMEM((TW, 16), jnp.float32),
            pltpu.SemaphoreType.DMA,
        ],
    )

    wbc, pos8, texp8 = _stage_a(x, gw, tri)
    posT = jnp.transpose(pos8[:, :TOPK])
    wb0 = wbc[:, :16]
    wb1 = wbc[:, 16:]
    texp = texp8[0, :NT]

    xs = dispatch(x, posT[0], posT[1])
    shared_out = _stage_shared(x, sw, sd)
    ys = _stage_c(texp, xs, ew, ed)
    out = combine(shared_out, ys, posT[0], posT[1], wb0, wb1)
    return out.reshape(b, s, d)


# speedup vs baseline: 1.6713x; 1.6594x over previous
import jax
import jax.numpy as jnp
from jax.experimental import pallas as pl

B, S, D = 1, 2048, 768
E, TOPK = 8, 2
I = 341
BT = 1024


def _moe_body(x_ref, gw_ref, sw_ref, sd_ref, ew_ref, ed_ref, out_ref):
    xt = x_ref[...]
    logits = jnp.dot(xt, gw_ref[...], preferred_element_type=jnp.float32)
    lane = jax.lax.broadcasted_iota(jnp.int32, (BT, E), 1)
    m = jnp.max(logits, axis=1, keepdims=True)
    ex = jnp.exp(logits - m)
    probs = ex / jnp.sum(ex, axis=1, keepdims=True)
    p1 = jnp.max(probs, axis=1, keepdims=True)
    sel1 = jnp.min(jnp.where(probs == p1, lane, E), axis=1, keepdims=True)
    probs2 = jnp.where(lane == sel1, -1.0, probs)
    p2 = jnp.max(probs2, axis=1, keepdims=True)
    sel2 = jnp.min(jnp.where(probs2 == p2, lane, E), axis=1, keepdims=True)
    wsum = p1 + p2 + 1e-8
    wsel = (jnp.where(lane == sel1, p1, 0.0)
            + jnp.where(lane == sel2, p2, 0.0)) / wsum

    xb = xt.astype(jnp.bfloat16)

    def ffn(w_gu, w_d):
        gu = jnp.dot(xb, w_gu, preferred_element_type=jnp.float32)
        g = gu[:, :I]
        u = gu[:, I:]
        h = (g * jax.nn.sigmoid(g)) * u
        return jnp.dot(h.astype(jnp.bfloat16), w_d,
                       preferred_element_type=jnp.float32)

    acc = ffn(sw_ref[...], sd_ref[...])
    for e in range(E):
        ye = ffn(ew_ref[e], ed_ref[e])
        acc = acc + wsel[:, e:e + 1] * ye
    out_ref[...] = acc


@jax.jit
def _moe(x, gw, sw, sd, ew, ed):
    grid = (S // BT,)
    return pl.pallas_call(
        _moe_body,
        grid=grid,
        in_specs=[
            pl.BlockSpec((BT, D), lambda i: (i, 0)),
            pl.BlockSpec((D, E), lambda i: (0, 0)),
            pl.BlockSpec((D, 2 * I), lambda i: (0, 0)),
            pl.BlockSpec((I, D), lambda i: (0, 0)),
            pl.BlockSpec((E, D, 2 * I), lambda i: (0, 0, 0)),
            pl.BlockSpec((E, I, D), lambda i: (0, 0, 0)),
        ],
        out_specs=pl.BlockSpec((BT, D), lambda i: (i, 0)),
        out_shape=jax.ShapeDtypeStruct((S, D), jnp.float32),
    )(x, gw, sw, sd, ew, ed)


def kernel(hidden_states, shared_gate_up_w, shared_down_w, expert_gate_up,
           expert_down, gate_weight):
    b, s, d = hidden_states.shape
    x = hidden_states.reshape(s, d)
    gw = gate_weight.T
    sw = shared_gate_up_w.T.astype(jnp.bfloat16)
    sd = shared_down_w.T.astype(jnp.bfloat16)
    ew = expert_gate_up.astype(jnp.bfloat16)
    ed = expert_down.astype(jnp.bfloat16)
    out = _moe(x, gw, sw, sd, ew, ed)
    return out.reshape(b, s, d)
